# baseline (speedup 1.0000x reference)
import functools
import numpy as np
import jax
import jax.numpy as jnp
from jax import lax
from jax.experimental import pallas as pl
from jax.experimental.pallas import tpu as pltpu
from jax.experimental.pallas import tpu_sc as plsc

_D = 64
_MAX_LEN = 512
_NUM_CORES = 2
_NUM_SUBCORES = 16
_NW = _NUM_CORES * _NUM_SUBCORES
_L16 = 16


def _make_pe2(max_len, d_model):
    position = np.arange(max_len, dtype=np.float32)[:, None]
    div_term = np.exp(
        np.arange(0, d_model, 2, dtype=np.float32) * -(np.log(10000.0) / d_model)
    )
    pe = np.zeros((max_len, d_model), dtype=np.float32)
    pe[:, 0::2] = np.sin(position * div_term)
    pe[:, 1::2] = np.cos(position * div_term)
    return pe * 2.0


_MESH = dict(
    core_axis_name="c", subcore_axis_name="s",
    num_cores=_NUM_CORES, num_subcores=_NUM_SUBCORES,
)
_TC_TILED = pltpu.CompilerParams(use_tc_tiling_on_sc=True,
                                 needs_layout_passes=False)


def _wid():
    return lax.axis_index("s") * _NUM_CORES + lax.axis_index("c")


def _detile_kernel(vocab):
    n_full = vocab // 128
    n_tailrows = (vocab - n_full * 128) // 2
    per_w = (n_full + _NW - 1) // _NW

    @functools.partial(
        pl.kernel,
        out_type=jax.ShapeDtypeStruct((vocab // 2, 128), jnp.float32),
        mesh=plsc.VectorSubcoreMesh(**_MESH),
        scratch_types=[pltpu.VMEM((8, 8, 128), jnp.float32),
                       pltpu.VMEM((64, 128), jnp.float32)],
        compiler_params=_TC_TILED,
    )
    def k1(tab_hbm, tail_hbm, out_hbm, buf, obuf):
        wid = _wid()
        iota = lax.iota(jnp.int32, _L16)
        d_lo = iota // 8
        d_md = iota % 8

        def do_col(vc):
            for d8 in range(8):
                pltpu.sync_copy(
                    tab_hbm.at[pl.ds(d8 * 8, 8), pl.ds(vc * 128, 128)],
                    buf.at[d8],
                )

            def row_body(r, carry):
                for j in range(8):
                    col = 2 * r + (1 if j >= 4 else 0)
                    colv = jnp.full((_L16,), 0, jnp.int32) + col
                    j4 = j % 4
                    g = plsc.load_gather(
                        buf, [2 * j4 + d_lo, d_md, colv])
                    obuf[r, pl.ds(j * 16, 16)] = g + g
                return carry

            lax.fori_loop(0, 64, row_body, 0)
            pltpu.sync_copy(obuf, out_hbm.at[pl.ds(vc * 64, 64)])

        def col_loop(i, carry):
            vc = wid + i * _NW

            @pl.when(vc < n_full)
            def _():
                do_col(vc)
            return carry

        lax.fori_loop(0, per_w, col_loop, 0)

        if n_tailrows:
            @pl.when(wid == 1 % _NW)
            def _():
                pltpu.sync_copy(tail_hbm, obuf.at[pl.ds(0, n_tailrows)])
                pltpu.sync_copy(obuf.at[pl.ds(0, n_tailrows)],
                                out_hbm.at[pl.ds(n_full * 64, n_tailrows)])

    return k1


def _gather_kernel(seq_len, batch, vocab):
    n_bc = batch // 128
    items = seq_len * n_bc
    per_w = items // _NW
    assert items % _NW == 0

    @functools.partial(
        pl.kernel,
        out_type=jax.ShapeDtypeStruct((seq_len, 64, batch), jnp.float32),
        mesh=plsc.VectorSubcoreMesh(**_MESH),
        scratch_types=[
            pltpu.VMEM((128,), jnp.int32),
            pltpu.VMEM((128,), jnp.int32),
            pltpu.VMEM((128,), jnp.int32),
            pltpu.VMEM((128, 128), jnp.float32),
            pltpu.VMEM((8, 128), jnp.float32),
            pltpu.VMEM((64, 128), jnp.float32),
            pltpu.SemaphoreType.DMA,
        ],
        compiler_params=_TC_TILED,
    )
    def k2(seq_hbm, tab_hbm, pe_hbm, out_hbm, idxv, rowv, hselv, gbuf,
           pebuf, obuf, sem):
        wid = _wid()
        iota = lax.iota(jnp.int32, _L16)

        def item_body(i, carry):
            item = wid + i * _NW
            l = item // n_bc
            bc = item % n_bc
            pltpu.sync_copy(seq_hbm.at[l, pl.ds(bc * 128, 128)], idxv)
            pltpu.sync_copy(pe_hbm.at[l], pebuf)
            for j in range(8):
                sl = pl.ds(j * 16, 16)
                v = idxv[sl]
                rowv[sl] = lax.shift_right_logical(v, 1)
                hselv[sl] = lax.bitwise_and(v, 1) * 64
            pltpu.async_copy(tab_hbm.at[rowv], gbuf, sem).wait()

            def d_body(d, c2):
                pe_d = pebuf[d // 8, pl.ds((d % 8) * 16, 16)]
                for jb in range(8):
                    sl = pl.ds(jb * 16, 16)
                    g = plsc.load_gather(gbuf, [jb * 16 + iota, hselv[sl] + d])
                    obuf[d, sl] = g + pe_d
                return c2

            lax.fori_loop(0, 64, d_body, 0)
            for d8 in range(8):
                pltpu.sync_copy(
                    obuf.at[pl.ds(d8 * 8, 8)],
                    out_hbm.at[l, pl.ds(d8 * 8, 8), pl.ds(bc * 128, 128)],
                )
            return carry

        lax.fori_loop(0, per_w, item_body, 0)

    return k2


@functools.partial(jax.jit, static_argnames=("batch", "seq_len", "vocab"))
def _embed(seqT, tableT, pe2b, batch, seq_len, vocab):
    n_tail = vocab % 128
    tail = (2.0 * tableT[:, vocab - n_tail:].T).reshape(n_tail // 2, 128)
    scratch = _detile_kernel(vocab)(tableT, tail)
    o3 = _gather_kernel(seq_len, batch, vocab)(seqT, scratch, pe2b)
    return jnp.transpose(o3, (2, 0, 1))


def kernel(sequence, table):
    batch, seq_len = sequence.shape
    vocab = table.shape[0]
    pe2 = _make_pe2(_MAX_LEN, _D)[:seq_len]
    pe2b = jnp.asarray(
        np.repeat(pe2, _L16, axis=1).reshape(seq_len, 8, 128))
    return _embed(sequence.T.astype(jnp.int32), table.T, pe2b,
                  batch, seq_len, vocab)


# speedup vs baseline: 1.3928x; 1.3928x over previous
import functools
import numpy as np
import jax
import jax.numpy as jnp
from jax import lax
from jax.experimental import pallas as pl
from jax.experimental.pallas import tpu as pltpu
from jax.experimental.pallas import tpu_sc as plsc

_D = 64
_MAX_LEN = 512
_NUM_CORES = 2
_NUM_SUBCORES = 16
_NW = _NUM_CORES * _NUM_SUBCORES
_L16 = 16


def _make_pe2(max_len, d_model):
    position = np.arange(max_len, dtype=np.float32)[:, None]
    div_term = np.exp(
        np.arange(0, d_model, 2, dtype=np.float32) * -(np.log(10000.0) / d_model)
    )
    pe = np.zeros((max_len, d_model), dtype=np.float32)
    pe[:, 0::2] = np.sin(position * div_term)
    pe[:, 1::2] = np.cos(position * div_term)
    return pe * 2.0


_MESH = dict(
    core_axis_name="c", subcore_axis_name="s",
    num_cores=_NUM_CORES, num_subcores=_NUM_SUBCORES,
)
_TC_TILED = pltpu.CompilerParams(use_tc_tiling_on_sc=True,
                                 needs_layout_passes=False)


def _wid():
    return lax.axis_index("s") * _NUM_CORES + lax.axis_index("c")


def _detile_kernel(vocab):
    ncol = 256
    n_it = (vocab // 128) // 2
    n_tailrows = (vocab % 128) // 2
    per_w = (n_it + _NW - 1) // _NW
    pad = 264

    @functools.partial(
        pl.kernel,
        out_type=jax.ShapeDtypeStruct((vocab // 2, 128), jnp.float32),
        mesh=plsc.VectorSubcoreMesh(**_MESH),
        scratch_types=[pltpu.VMEM((64, pad), jnp.float32),
                       pltpu.VMEM((128, 128), jnp.float32),
                       pltpu.SemaphoreType.DMA],
        compiler_params=_TC_TILED,
    )
    def k1(tab_hbm, tail_hbm, out_hbm, buf, obuf, sem):
        wid = _wid()
        iota = lax.iota(jnp.int32, _L16)

        def it_body(i, carry):
            it = wid + i * _NW

            @pl.when(it < n_it)
            def _():
                c0 = it * ncol
                pltpu.async_copy(tab_hbm.at[:, pl.ds(c0, ncol)],
                                 buf.at[:, pl.ds(0, ncol)], sem).wait()

                def row_body(r4, c2):
                    for rr in range(4):
                        r = r4 * 4 + rr
                        for j in range(8):
                            col = 2 * r + (1 if j >= 4 else 0)
                            colv = jnp.full((_L16,), 0, jnp.int32) + col
                            dv = (j % 4) * 16 + iota
                            g = plsc.load_gather(buf, [dv, colv])
                            obuf[r, pl.ds(j * 16, 16)] = g + g
                    return c2

                lax.fori_loop(0, 32, row_body, 0)
                pltpu.sync_copy(obuf, out_hbm.at[pl.ds(it * 128, 128)])
            return carry

        lax.fori_loop(0, per_w, it_body, 0)

        if n_tailrows:
            @pl.when(wid == 1 % _NW)
            def _():
                pltpu.sync_copy(tail_hbm, obuf.at[pl.ds(0, n_tailrows)])
                pltpu.sync_copy(obuf.at[pl.ds(0, n_tailrows)],
                                out_hbm.at[pl.ds(n_it * 128, n_tailrows)])

    return k1


def _gather_kernel(seq_len, batch, vocab):
    n_bc = batch // 128
    items = seq_len * n_bc
    per_w = items // _NW
    assert items % _NW == 0
    gpad = 136

    @functools.partial(
        pl.kernel,
        out_type=jax.ShapeDtypeStruct((seq_len, 64, batch), jnp.float32),
        mesh=plsc.VectorSubcoreMesh(**_MESH),
        scratch_types=[
            pltpu.VMEM((128,), jnp.int32),
            pltpu.VMEM((2, 1, 128), jnp.int32),
            pltpu.VMEM((2, 128), jnp.int32),
            pltpu.VMEM((128, gpad), jnp.float32),
            pltpu.VMEM((128, gpad), jnp.float32),
            pltpu.VMEM((8, 128), jnp.float32),
            pltpu.VMEM((64, 128), jnp.float32),
            pltpu.SemaphoreType.DMA,
            pltpu.SemaphoreType.DMA,
        ],
        compiler_params=_TC_TILED,
    )
    def k2(seq_hbm, tab_hbm, pe_hbm, out_hbm, idxv, rowv, hselv, gbuf0,
           gbuf1, pebuf, obuf, sem0, sem1):
        wid = _wid()
        iota = lax.iota(jnp.int32, _L16)
        gbufs = (gbuf0, gbuf1)
        sems = (sem0, sem1)

        def fire(item, par):
            l = item // n_bc
            bc = item % n_bc
            pltpu.sync_copy(seq_hbm.at[l, pl.ds(bc * 128, 128)], idxv)
            for j in range(8):
                sl = pl.ds(j * 16, 16)
                v = idxv[sl]
                rowv[par, 0, sl] = lax.shift_right_logical(v, 1)
                hselv[par, sl] = lax.bitwise_and(v, 1) * 64
            pltpu.async_copy(tab_hbm.at[rowv.at[par, 0]],
                             gbufs[par].at[:, pl.ds(0, 128)], sems[par])

        def consume(item, par):
            gbuf = gbufs[par]
            l = item // n_bc
            bc = item % n_bc
            pltpu.sync_copy(pe_hbm.at[l], pebuf)
            pltpu.make_async_copy(
                tab_hbm.at[rowv.at[par, 0]],
                gbuf.at[:, pl.ds(0, 128)], sems[par]).wait()

            def d_body(d4, c2):
                for dd in range(4):
                    d = d4 * 4 + dd
                    pe_d = pebuf[d // 8, pl.ds((d % 8) * 16, 16)]
                    for jb in range(8):
                        sl = pl.ds(jb * 16, 16)
                        g = plsc.load_gather(
                            gbuf, [jb * 16 + iota, hselv[par, sl] + d])
                        obuf[d, sl] = g + pe_d
                return c2

            lax.fori_loop(0, 16, d_body, 0)
            pltpu.sync_copy(obuf,
                            out_hbm.at[l, :, pl.ds(bc * 128, 128)])

        fire(wid, 0)

        def item_body(i, carry):
            item = wid + i * _NW

            @pl.when(i % 2 == 0)
            def _():
                @pl.when(i + 1 < per_w)
                def _():
                    fire(item + _NW, 1)
                consume(item, 0)

            @pl.when(i % 2 == 1)
            def _():
                @pl.when(i + 1 < per_w)
                def _():
                    fire(item + _NW, 0)
                consume(item, 1)
            return carry

        lax.fori_loop(0, per_w, item_body, 0)

    return k2


@functools.partial(jax.jit, static_argnames=("batch", "seq_len", "vocab"))
def _embed(seqT, tableT, pe2b, batch, seq_len, vocab):
    n_tail = vocab % 128
    tail = (2.0 * tableT[:, vocab - n_tail:].T).reshape(n_tail // 2, 128)
    scratch = _detile_kernel(vocab)(tableT, tail)
    o3 = _gather_kernel(seq_len, batch, vocab)(seqT, scratch, pe2b)
    return jnp.transpose(o3, (2, 0, 1))


def kernel(sequence, table):
    batch, seq_len = sequence.shape
    vocab = table.shape[0]
    pe2 = _make_pe2(_MAX_LEN, _D)[:seq_len]
    pe2b = jnp.asarray(
        np.repeat(pe2, _L16, axis=1).reshape(seq_len, 8, 128))
    return _embed(sequence.T.astype(jnp.int32), table.T, pe2b,
                  batch, seq_len, vocab)


# speedup vs baseline: 9.3534x; 6.7154x over previous
import functools
import numpy as np
import jax
import jax.numpy as jnp
from jax import lax
from jax.experimental import pallas as pl
from jax.experimental.pallas import tpu as pltpu
from jax.experimental.pallas import tpu_sc as plsc

_D = 64
_MAX_LEN = 512
_NUM_CORES = 2
_NUM_SUBCORES = 16
_NW = _NUM_CORES * _NUM_SUBCORES
_L16 = 16


def _make_pe2(max_len, d_model):
    position = np.arange(max_len, dtype=np.float32)[:, None]
    div_term = np.exp(
        np.arange(0, d_model, 2, dtype=np.float32) * -(np.log(10000.0) / d_model)
    )
    pe = np.zeros((max_len, d_model), dtype=np.float32)
    pe[:, 0::2] = np.sin(position * div_term)
    pe[:, 1::2] = np.cos(position * div_term)
    return pe * 2.0


_MESH = dict(
    core_axis_name="c", subcore_axis_name="s",
    num_cores=_NUM_CORES, num_subcores=_NUM_SUBCORES,
)
_TC_TILED = pltpu.CompilerParams(use_tc_tiling_on_sc=True,
                                 needs_layout_passes=False,
                                 disable_bounds_checks=True)


def _wid():
    return lax.axis_index("s") * _NUM_CORES + lax.axis_index("c")


def _detile_kernel(vocab):
    ncol = 256
    n_it = (vocab // 128) // 2
    n_tailrows = (vocab % 128) // 2
    per_w = (n_it + _NW - 1) // _NW
    pad = 264

    @functools.partial(
        pl.kernel,
        out_type=jax.ShapeDtypeStruct((vocab // 2, 128), jnp.float32),
        mesh=plsc.VectorSubcoreMesh(**_MESH),
        scratch_types=[pltpu.VMEM((64, pad), jnp.float32),
                       pltpu.VMEM((128, 128), jnp.float32),
                       pltpu.SemaphoreType.DMA],
        compiler_params=_TC_TILED,
    )
    def k1(tab_hbm, tail_hbm, out_hbm, buf, obuf, sem):
        wid = _wid()
        iota = lax.iota(jnp.int32, _L16)

        def it_body(i, carry):
            it = wid + i * _NW

            @pl.when(it < n_it)
            def _():
                c0 = it * ncol
                pltpu.async_copy(tab_hbm.at[:, pl.ds(c0, ncol)],
                                 buf.at[:, pl.ds(0, ncol)], sem).wait()

                @functools.partial(plsc.parallel_loop, 0, 32, unroll=4)
                def row_body(r4):
                    for rr in range(4):
                        r = r4 * 4 + rr
                        for j in range(8):
                            col = 2 * r + (1 if j >= 4 else 0)
                            colv = jnp.full((_L16,), 0, jnp.int32) + col
                            dv = (j % 4) * 16 + iota
                            g = plsc.load_gather(buf, [dv, colv])
                            obuf[r, pl.ds(j * 16, 16)] = g + g
                pltpu.sync_copy(obuf, out_hbm.at[pl.ds(it * 128, 128)])
            return carry

        lax.fori_loop(0, per_w, it_body, 0)

        if n_tailrows:
            @pl.when(wid == 1 % _NW)
            def _():
                pltpu.sync_copy(tail_hbm, obuf.at[pl.ds(0, n_tailrows)])
                pltpu.sync_copy(obuf.at[pl.ds(0, n_tailrows)],
                                out_hbm.at[pl.ds(n_it * 128, n_tailrows)])

    return k1


def _gather_kernel(seq_len, batch, vocab):
    n_bc = batch // 128
    items = seq_len * n_bc
    per_w = items // _NW
    assert items % _NW == 0
    gpad = 136

    @functools.partial(
        pl.kernel,
        out_type=jax.ShapeDtypeStruct((seq_len, 64, batch), jnp.float32),
        mesh=plsc.VectorSubcoreMesh(**_MESH),
        scratch_types=[
            pltpu.VMEM((128,), jnp.int32),
            pltpu.VMEM((2, 1, 128), jnp.int32),
            pltpu.VMEM((2, 128), jnp.int32),
            pltpu.VMEM((128, gpad), jnp.float32),
            pltpu.VMEM((128, gpad), jnp.float32),
            pltpu.VMEM((8, 128), jnp.float32),
            pltpu.VMEM((64, 128), jnp.float32),
            pltpu.SemaphoreType.DMA,
            pltpu.SemaphoreType.DMA,
        ],
        compiler_params=_TC_TILED,
    )
    def k2(seq_hbm, tab_hbm, pe_hbm, out_hbm, idxv, rowv, hselv, gbuf0,
           gbuf1, pebuf, obuf, sem0, sem1):
        wid = _wid()
        iota = lax.iota(jnp.int32, _L16)
        gbufs = (gbuf0, gbuf1)
        sems = (sem0, sem1)

        def fire(item, par):
            l = item // n_bc
            bc = item % n_bc
            pltpu.sync_copy(seq_hbm.at[l, pl.ds(bc * 128, 128)], idxv)
            for j in range(8):
                sl = pl.ds(j * 16, 16)
                v = idxv[sl]
                rowv[par, 0, sl] = lax.shift_right_logical(v, 1)
                hselv[par, sl] = lax.bitwise_and(v, 1) * 64
            pltpu.async_copy(tab_hbm.at[rowv.at[par, 0]],
                             gbufs[par].at[:, pl.ds(0, 128)], sems[par])

        def consume(item, par):
            gbuf = gbufs[par]
            l = item // n_bc
            bc = item % n_bc
            pltpu.sync_copy(pe_hbm.at[l], pebuf)
            pltpu.make_async_copy(
                tab_hbm.at[rowv.at[par, 0]],
                gbuf.at[:, pl.ds(0, 128)], sems[par]).wait()

            @functools.partial(plsc.parallel_loop, 0, 16, unroll=4)
            def d_body(d4):
                for dd in range(4):
                    d = d4 * 4 + dd
                    pe_d = pebuf[d // 8, pl.ds((d % 8) * 16, 16)]
                    for jb in range(8):
                        sl = pl.ds(jb * 16, 16)
                        g = plsc.load_gather(
                            gbuf, [jb * 16 + iota, hselv[par, sl] + d])
                        obuf[d, sl] = g + pe_d
            pltpu.sync_copy(obuf,
                            out_hbm.at[l, :, pl.ds(bc * 128, 128)])

        fire(wid, 0)

        def item_body(i, carry):
            item = wid + i * _NW

            @pl.when(i % 2 == 0)
            def _():
                @pl.when(i + 1 < per_w)
                def _():
                    fire(item + _NW, 1)
                consume(item, 0)

            @pl.when(i % 2 == 1)
            def _():
                @pl.when(i + 1 < per_w)
                def _():
                    fire(item + _NW, 0)
                consume(item, 1)
            return carry

        lax.fori_loop(0, per_w, item_body, 0)

    return k2


@functools.partial(jax.jit, static_argnames=("batch", "seq_len", "vocab"))
def _embed(seqT, tableT, pe2b, batch, seq_len, vocab):
    n_tail = vocab % 128
    tail = (2.0 * tableT[:, vocab - n_tail:].T).reshape(n_tail // 2, 128)
    scratch = _detile_kernel(vocab)(tableT, tail)
    o3 = _gather_kernel(seq_len, batch, vocab)(seqT, scratch, pe2b)
    return jnp.transpose(o3, (2, 0, 1))


def kernel(sequence, table):
    batch, seq_len = sequence.shape
    vocab = table.shape[0]
    pe2 = _make_pe2(_MAX_LEN, _D)[:seq_len]
    pe2b = jnp.asarray(
        np.repeat(pe2, _L16, axis=1).reshape(seq_len, 8, 128))
    return _embed(sequence.T.astype(jnp.int32), table.T, pe2b,
                  batch, seq_len, vocab)
